# Initial kernel scaffold; baseline (speedup 1.0000x reference)
import jax
import jax.numpy as jnp
from jax.experimental import pallas as pl


def kernel(g, g_r, g_n, si, sj, sn, e, t, vi, vj, vn, Wdu, bdu, Wdv, bdv, Wm, bm, Wq, Wk, Wvp, bq, bk, bvp, Wo, bo, basis_freq, phase):
    raise NotImplementedError("write your pallas kernel here")


# speedup vs baseline: 1.6683x; 1.6683x over previous
import functools

import jax
import jax.numpy as jnp
from jax.experimental import pallas as pl

S = 128
T = 112
EF = 16
BLK = 1000


def _post_body(t0_ref, h_ref, hsum_ref, esum_ref, csum_ref, ssum_ref,
               w1_ref, w2_ref, w3_ref, w4_ref, bc_ref, f_ref, p_ref, out_ref):
    t0 = t0_ref[...]
    ang = t0 * f_ref[...] + p_ref[...]
    tsum = jnp.cos(ang) * csum_ref[...] + jnp.sin(ang) * ssum_ref[...]
    acc = jnp.dot(h_ref[...], w1_ref[...], preferred_element_type=jnp.float32)
    acc += jnp.dot(hsum_ref[...], w2_ref[...], preferred_element_type=jnp.float32)
    acc += jnp.dot(esum_ref[...], w3_ref[...], preferred_element_type=jnp.float32)
    acc += jnp.dot(tsum, w4_ref[...], preferred_element_type=jnp.float32)
    out_ref[...] = acc + bc_ref[...]


def _post(t0, h, hsum, esum, csum, ssum, w1, w2, w3, w4, bc, f, p, n):
    grid = (n // BLK,)
    full = lambda r, c: pl.BlockSpec((None, r, c), lambda i: (0, 0, 0))
    row = lambda c: pl.BlockSpec((BLK, c), lambda i: (i, 0))
    return pl.pallas_call(
        _post_body,
        grid=grid,
        in_specs=[
            pl.BlockSpec((BLK, 1), lambda i: (i, 0)),
            row(S), row(S), row(EF), row(T), row(T),
            pl.BlockSpec((S, S), lambda i: (0, 0)),
            pl.BlockSpec((S, S), lambda i: (0, 0)),
            pl.BlockSpec((EF, S), lambda i: (0, 0)),
            pl.BlockSpec((T, S), lambda i: (0, 0)),
            pl.BlockSpec((1, S), lambda i: (0, 0)),
            pl.BlockSpec((1, T), lambda i: (0, 0)),
            pl.BlockSpec((1, T), lambda i: (0, 0)),
        ],
        out_specs=row(S),
        out_shape=jax.ShapeDtypeStruct((n, S), jnp.float32),
    )(t0[:, None], h, hsum, esum, csum, ssum, w1, w2, w3, w4,
      bc[None, :], f[None, :], p[None, :])


def kernel(g, g_r, g_n, si, sj, sn, e, t, vi, vj, vn, Wdu, bdu, Wdv, bdv,
           Wm, bm, Wq, Wk, Wvp, bq, bk, bvp, Wo, bo, basis_freq, phase):
    NU = si.shape[0]
    NI = sj.shape[0]

    Wm1 = Wm[:, :S]
    Wm2 = Wm[:, S:]
    Wc = Wvp.T @ Wo.T @ Wm2.T
    bc = (bvp @ Wo.T + bo) @ Wm2.T + bm
    w1 = Wm1.T
    w2, w3, w4 = Wc[:S], Wc[S:S + EF], Wc[S + EF:]

    hi = vi @ Wdu.T + bdu + si
    hj = vj @ Wdv.T + bdv + sj
    hn = vn @ Wdv.T + bdv + sn

    ce = jnp.cos(t[:, None] * basis_freq[None, :])
    se = jnp.sin(t[:, None] * basis_freq[None, :])

    def agg(gg, h_src, n_dst):
        src, dst = gg[0], gg[1]
        t0 = jax.ops.segment_max(t, dst, num_segments=n_dst)
        t0 = jnp.where(jnp.isfinite(t0), t0, 0.0)
        payload = jnp.concatenate([h_src[src], e, ce, se], axis=1)
        sums = jax.ops.segment_sum(payload, dst, num_segments=n_dst)
        return t0, sums

    def out(gg, h_src, h_dst, n_dst):
        t0, sums = agg(gg, h_src, n_dst)
        hsum = sums[:, :S]
        esum = sums[:, S:S + EF]
        csum = sums[:, S + EF:S + EF + T]
        ssum = sums[:, S + EF + T:]
        return _post(t0, h_dst, hsum, esum, csum, ssum,
                     w1, w2, w3, w4, bc, basis_freq, phase, n_dst)

    hj_o = out(g, hi, hj, NI)
    hi_o = out(g_r, hj, hi, NU)
    hn_o = out(g_n, hi, hn, NI)
    return (hi_o, hj_o, hn_o)
